# baseline (speedup 1.0000x reference)
import functools

import jax
import jax.numpy as jnp
from jax import lax
from jax.experimental import pallas as pl
from jax.experimental.pallas import tpu as pltpu
from jax.experimental.pallas import tpu_sc as plsc

N_NODES = 100000
D = 128
B = 4096
N_NEG = 20
N_PAIR = N_NEG + 1
N_SPLIT = 2
BH = B // N_SPLIT

NC = 2
NS = 16
NW = NC * NS
L = 16

EDGES_PER_CHUNK = 4
CHUNK_NEG_ROWS = EDGES_PER_CHUNK * N_NEG

RES_LANES = 512
RES_ROWS = 2 * EDGES_PER_CHUNK


def _sc_pair_partials(table, xs, ys, y_neg_flat, nb):
    edges_per_tile = nb // NW
    chunks_per_tile = edges_per_tile // EDGES_PER_CHUNK
    neg_per_tile = edges_per_tile * N_NEG
    mesh = plsc.VectorSubcoreMesh(core_axis_name="c", subcore_axis_name="s")

    @functools.partial(
        pl.kernel,
        out_type=jax.ShapeDtypeStruct((nb, RES_LANES), jnp.float32),
        mesh=mesh,
        scratch_types=[
            pltpu.VMEM((edges_per_tile,), jnp.int32),
            pltpu.VMEM((edges_per_tile,), jnp.int32),
            pltpu.VMEM((neg_per_tile,), jnp.int32),
            pltpu.VMEM((2 * edges_per_tile, D), jnp.float32),
            pltpu.VMEM((CHUNK_NEG_ROWS, D), jnp.float32),
            pltpu.VMEM((CHUNK_NEG_ROWS, D), jnp.float32),
            pltpu.VMEM((RES_ROWS, RES_LANES), jnp.float32),
            pltpu.SemaphoreType.DMA,
            pltpu.SemaphoreType.DMA,
            pltpu.SemaphoreType.DMA,
        ],
    )
    def sc_kernel(table_hbm, xs_hbm, ys_hbm, yneg_hbm, res_hbm,
                  xi_v, yi_v, ni_v, xy_v, neg0, neg1, res_v,
                  sem_xy, sem_g, sem_w):
        wid = lax.axis_index("s") * NC + lax.axis_index("c")
        edge_base = wid * edges_per_tile

        pltpu.sync_copy(xs_hbm.at[pl.ds(edge_base, edges_per_tile)], xi_v)
        pltpu.sync_copy(ys_hbm.at[pl.ds(edge_base, edges_per_tile)], yi_v)
        pltpu.sync_copy(yneg_hbm.at[pl.ds(wid * neg_per_tile, neg_per_tile)],
                        ni_v)

        pltpu.async_copy(table_hbm.at[xi_v],
                         xy_v.at[pl.ds(0, edges_per_tile)], sem_xy)
        pltpu.async_copy(table_hbm.at[yi_v],
                         xy_v.at[pl.ds(edges_per_tile, edges_per_tile)],
                         sem_xy)

        def start_neg_gather(chunk, buf):
            return pltpu.async_copy(
                table_hbm.at[ni_v.at[pl.ds(chunk * CHUNK_NEG_ROWS,
                                           CHUNK_NEG_ROWS)]],
                buf, sem_g)

        start_neg_gather(0, neg0)
        start_neg_gather(1, neg1)

        zeros = jnp.zeros((L,), jnp.float32)
        for r in range(RES_ROWS):
            for c in range(RES_LANES // L):
                res_v[r, pl.ds(c * L, L)] = zeros

        pltpu.make_async_copy(table_hbm.at[xi_v],
                              xy_v.at[pl.ds(0, edges_per_tile)],
                              sem_xy).wait()
        pltpu.make_async_copy(table_hbm.at[xi_v],
                              xy_v.at[pl.ds(0, edges_per_tile)],
                              sem_xy).wait()

        def compute_chunk(t, chunk_parity, neg_v, res_row_base):
            for e in range(EDGES_PER_CHUNK):
                edge = (2 * t + chunk_parity) * EDGES_PER_CHUNK + e
                res_row = res_row_base + e
                xq = [xy_v[edge, pl.ds(k * L, L)] for k in range(D // L)]
                for p in range(N_NEG):
                    prow = e * N_NEG + p
                    acc = None
                    for k in range(D // L):
                        dlt = xq[k] - neg_v[prow, pl.ds(k * L, L)]
                        sq = dlt * dlt
                        acc = sq if acc is None else acc + sq
                    res_v[res_row, pl.ds(p * L, L)] = acc
                acc = None
                for k in range(D // L):
                    dlt = xq[k] - xy_v[edges_per_tile + edge, pl.ds(k * L, L)]
                    sq = dlt * dlt
                    acc = sq if acc is None else acc + sq
                res_v[res_row, pl.ds(N_NEG * L, L)] = acc

        @pl.loop(0, chunks_per_tile // 2)
        def _(t):
            c0 = 2 * t

            @pl.when(t > 0)
            def _():
                pltpu.make_async_copy(
                    res_v, res_hbm.at[pl.ds(edge_base, RES_ROWS)], sem_w
                ).wait()

            pltpu.make_async_copy(
                table_hbm.at[ni_v.at[pl.ds(c0 * CHUNK_NEG_ROWS,
                                           CHUNK_NEG_ROWS)]],
                neg0, sem_g).wait()
            compute_chunk(t, 0, neg0, 0)

            @pl.when(t < chunks_per_tile // 2 - 1)
            def _():
                start_neg_gather(c0 + 2, neg0)

            pltpu.make_async_copy(
                table_hbm.at[ni_v.at[pl.ds((c0 + 1) * CHUNK_NEG_ROWS,
                                           CHUNK_NEG_ROWS)]],
                neg1, sem_g).wait()
            compute_chunk(t, 1, neg1, EDGES_PER_CHUNK)

            @pl.when(t < chunks_per_tile // 2 - 1)
            def _():
                start_neg_gather(c0 + 3, neg1)

            pltpu.async_copy(
                res_v,
                res_hbm.at[pl.ds(edge_base + t * RES_ROWS, RES_ROWS)],
                sem_w)

        pltpu.make_async_copy(
            res_v, res_hbm.at[pl.ds(edge_base, RES_ROWS)], sem_w).wait()

    return sc_kernel(table, xs, ys, y_neg_flat)


def _tc_loss_body(r_ref, w_ref, o_ref):
    blk = r_ref[...]

    row_ids = lax.broadcasted_iota(jnp.int32, (RES_LANES, RES_LANES // L), 0)
    col_ids = lax.broadcasted_iota(jnp.int32, (RES_LANES, RES_LANES // L), 1)
    sel = (row_ids // L == col_ids).astype(jnp.float32)

    dmat = jax.lax.dot_general(
        blk, sel, (((1,), (0,)), ((), ())),
        preferred_element_type=jnp.float32)

    dneg = dmat[:, :N_NEG]
    dpos = dmat[:, N_NEG:N_NEG + 1]

    p_pos = jnp.clip(1.0 / (1.0 + 0.25 * dpos), 1e-12, 1.0 - 1e-12)
    p_neg = jnp.clip(1.0 / (1.0 + 0.25 * dneg), 1e-12, 0.99)
    t_pos = jnp.log(p_pos) * 20.0
    t_neg = jnp.log(1.0 - p_neg)

    loss_b = 7.0 * jnp.sum(t_neg, axis=1) + t_pos[:, 0]
    o_ref[...] = jnp.full((1, 1), -jnp.sum(w_ref[0, :] * loss_b), jnp.float32)


def _tc_loss(pair_partials, weights):
    out = pl.pallas_call(
        _tc_loss_body,
        out_shape=jax.ShapeDtypeStruct((1, 1), jnp.float32),
    )(pair_partials, weights)
    return out[0, 0]


@jax.jit
def kernel(logits, xs, ys, y_neg, sample_edge_weight):
    y_neg_flat = y_neg.reshape(-1)
    total = jnp.float32(0)
    for h in range(N_SPLIT):
        lo, nlo = h * BH, h * BH * N_NEG
        partials = _sc_pair_partials(
            logits, xs[lo:lo + BH], ys[lo:lo + BH],
            y_neg_flat[nlo:nlo + BH * N_NEG], BH)
        total = total + _tc_loss(
            partials, sample_edge_weight[lo:lo + BH].reshape(1, BH))
    return total


# speedup vs baseline: 1.1269x; 1.1269x over previous
import functools

import jax
import jax.numpy as jnp
from jax import lax
from jax.experimental import pallas as pl
from jax.experimental.pallas import tpu as pltpu
from jax.experimental.pallas import tpu_sc as plsc

N_NODES = 100000
D = 128
B = 4096
N_NEG = 20
N_PAIR = N_NEG + 1

NC = 2
NS = 16
NW = NC * NS
L = 16

EDGES_PER_TILE = B // NW
EDGES_PER_CHUNK = 4
CHUNK_NEG_ROWS = EDGES_PER_CHUNK * N_NEG
CHUNKS_PER_TILE = EDGES_PER_TILE // EDGES_PER_CHUNK
NEG_PER_TILE = EDGES_PER_TILE * N_NEG
NBUF = 2

RES_LANES = 384
RES_ROWS = NBUF * EDGES_PER_CHUNK


def _sc_pair_partials(table, xs, ys, y_neg_flat):
    mesh = plsc.VectorSubcoreMesh(core_axis_name="c", subcore_axis_name="s")

    @functools.partial(
        pl.kernel,
        out_type=jax.ShapeDtypeStruct((B, RES_LANES), jnp.float32),
        mesh=mesh,
        scratch_types=[
            pltpu.VMEM((EDGES_PER_TILE,), jnp.int32),
            pltpu.VMEM((EDGES_PER_TILE,), jnp.int32),
            pltpu.VMEM((NEG_PER_TILE,), jnp.int32),
            pltpu.VMEM((2 * EDGES_PER_TILE, D), jnp.float32),
            pltpu.VMEM((CHUNK_NEG_ROWS, D), jnp.float32),
            pltpu.VMEM((CHUNK_NEG_ROWS, D), jnp.float32),
            pltpu.VMEM((RES_ROWS, RES_LANES), jnp.float32),
            pltpu.SemaphoreType.DMA,
            pltpu.SemaphoreType.DMA,
            pltpu.SemaphoreType.DMA,
            pltpu.SemaphoreType.DMA,
        ],
    )
    def sc_kernel(table_hbm, xs_hbm, ys_hbm, yneg_hbm, res_hbm,
                  xi_v, yi_v, ni_v, xy_v, neg0, neg1, res_v,
                  sem_xy, sem_g0, sem_g1, sem_w):
        wid = lax.axis_index("s") * NC + lax.axis_index("c")
        edge_base = wid * EDGES_PER_TILE
        negs = (neg0, neg1)
        sems = (sem_g0, sem_g1)

        pltpu.sync_copy(xs_hbm.at[pl.ds(edge_base, EDGES_PER_TILE)], xi_v)
        pltpu.sync_copy(ys_hbm.at[pl.ds(edge_base, EDGES_PER_TILE)], yi_v)
        pltpu.sync_copy(yneg_hbm.at[pl.ds(wid * NEG_PER_TILE, NEG_PER_TILE)],
                        ni_v)

        pltpu.async_copy(table_hbm.at[xi_v],
                         xy_v.at[pl.ds(0, EDGES_PER_TILE)], sem_xy)
        pltpu.async_copy(table_hbm.at[yi_v],
                         xy_v.at[pl.ds(EDGES_PER_TILE, EDGES_PER_TILE)],
                         sem_xy)

        def start_neg_gather(chunk, j):
            return pltpu.async_copy(
                table_hbm.at[ni_v.at[pl.ds(chunk * CHUNK_NEG_ROWS,
                                           CHUNK_NEG_ROWS)]],
                negs[j], sems[j])

        for j in range(NBUF):
            start_neg_gather(j, j)

        zeros = jnp.zeros((L,), jnp.float32)
        for r in range(RES_ROWS):
            for c in range(RES_LANES // L):
                res_v[r, pl.ds(c * L, L)] = zeros

        pltpu.make_async_copy(table_hbm.at[xi_v],
                              xy_v.at[pl.ds(0, EDGES_PER_TILE)],
                              sem_xy).wait()
        pltpu.make_async_copy(table_hbm.at[xi_v],
                              xy_v.at[pl.ds(0, EDGES_PER_TILE)],
                              sem_xy).wait()

        def compute_chunk(t, j, neg_v):
            for e in range(EDGES_PER_CHUNK):
                edge = (NBUF * t + j) * EDGES_PER_CHUNK + e
                res_row = j * EDGES_PER_CHUNK + e
                xq = [xy_v[edge, pl.ds(k * L, L)] for k in range(D // L)]
                for p in range(N_NEG):
                    prow = e * N_NEG + p
                    acc = None
                    for k in range(D // L):
                        dlt = xq[k] - neg_v[prow, pl.ds(k * L, L)]
                        sq = dlt * dlt
                        acc = sq if acc is None else acc + sq
                    res_v[res_row, pl.ds(p * L, L)] = acc
                acc = None
                for k in range(D // L):
                    dlt = xq[k] - xy_v[EDGES_PER_TILE + edge, pl.ds(k * L, L)]
                    sq = dlt * dlt
                    acc = sq if acc is None else acc + sq
                res_v[res_row, pl.ds(N_NEG * L, L)] = acc

        @pl.loop(0, CHUNKS_PER_TILE // NBUF)
        def _(t):
            @pl.when(t > 0)
            def _():
                pltpu.make_async_copy(
                    res_v, res_hbm.at[pl.ds(edge_base, RES_ROWS)], sem_w
                ).wait()

            for j in range(NBUF):
                chunk = NBUF * t + j
                pltpu.make_async_copy(
                    table_hbm.at[ni_v.at[pl.ds(chunk * CHUNK_NEG_ROWS,
                                               CHUNK_NEG_ROWS)]],
                    negs[j], sems[j]).wait()
                compute_chunk(t, j, negs[j])

                @pl.when(t < CHUNKS_PER_TILE // NBUF - 1)
                def _():
                    start_neg_gather(chunk + NBUF, j)

            pltpu.async_copy(
                res_v,
                res_hbm.at[pl.ds(edge_base + t * RES_ROWS, RES_ROWS)],
                sem_w)

        pltpu.make_async_copy(
            res_v, res_hbm.at[pl.ds(edge_base, RES_ROWS)], sem_w).wait()

    return sc_kernel(table, xs, ys, y_neg_flat)


def _tc_loss_body(r_ref, w_ref, o_ref):
    blk = r_ref[...]

    row_ids = lax.broadcasted_iota(jnp.int32, (RES_LANES, RES_LANES // L), 0)
    col_ids = lax.broadcasted_iota(jnp.int32, (RES_LANES, RES_LANES // L), 1)
    sel = (row_ids // L == col_ids).astype(jnp.float32)

    dmat = jax.lax.dot_general(
        blk, sel, (((1,), (0,)), ((), ())),
        preferred_element_type=jnp.float32)

    dneg = dmat[:, :N_NEG]
    dpos = dmat[:, N_NEG:N_NEG + 1]

    p_pos = jnp.clip(1.0 / (1.0 + 0.25 * dpos), 1e-12, 1.0 - 1e-12)
    p_neg = jnp.clip(1.0 / (1.0 + 0.25 * dneg), 1e-12, 0.99)
    t_pos = jnp.log(p_pos) * 20.0
    t_neg = jnp.log(1.0 - p_neg)

    loss_b = 7.0 * jnp.sum(t_neg, axis=1) + t_pos[:, 0]
    o_ref[...] = jnp.full((1, 1), -jnp.sum(w_ref[0, :] * loss_b), jnp.float32)


def _tc_loss(pair_partials, weights):
    out = pl.pallas_call(
        _tc_loss_body,
        out_shape=jax.ShapeDtypeStruct((1, 1), jnp.float32),
    )(pair_partials, weights)
    return out[0, 0]


@jax.jit
def kernel(logits, xs, ys, y_neg, sample_edge_weight):
    partials = _sc_pair_partials(logits, xs, ys, y_neg.reshape(-1))
    return _tc_loss(partials, sample_edge_weight.reshape(1, B))


# speedup vs baseline: 1.1883x; 1.0545x over previous
import functools

import jax
import jax.numpy as jnp
from jax import lax
from jax.experimental import pallas as pl
from jax.experimental.pallas import tpu as pltpu
from jax.experimental.pallas import tpu_sc as plsc

N_NODES = 100000
D = 128
B = 4096
N_NEG = 20
N_PAIR = N_NEG + 1

NC = 2
NS = 16
NW = NC * NS
L = 16

EDGES_PER_TILE = B // NW
EDGES_PER_CHUNK = 2
CHUNK_NEG_ROWS = EDGES_PER_CHUNK * N_NEG
CHUNKS_PER_TILE = EDGES_PER_TILE // EDGES_PER_CHUNK
NEG_PER_TILE = EDGES_PER_TILE * N_NEG
NBUF = 4

RES_LANES = 384
RES_ROWS = NBUF * EDGES_PER_CHUNK


def _sc_pair_partials(table, xs, ys, y_neg_flat):
    mesh = plsc.VectorSubcoreMesh(core_axis_name="c", subcore_axis_name="s")

    @functools.partial(
        pl.kernel,
        out_type=jax.ShapeDtypeStruct((B, RES_LANES), jnp.float32),
        mesh=mesh,
        scratch_types=[
            pltpu.VMEM((EDGES_PER_TILE,), jnp.int32),
            pltpu.VMEM((EDGES_PER_TILE,), jnp.int32),
            pltpu.VMEM((NEG_PER_TILE,), jnp.int32),
            pltpu.VMEM((2 * EDGES_PER_TILE, D), jnp.float32),
            pltpu.VMEM((CHUNK_NEG_ROWS, D), jnp.float32),
            pltpu.VMEM((CHUNK_NEG_ROWS, D), jnp.float32),
            pltpu.VMEM((CHUNK_NEG_ROWS, D), jnp.float32),
            pltpu.VMEM((CHUNK_NEG_ROWS, D), jnp.float32),
            pltpu.VMEM((RES_ROWS, RES_LANES), jnp.float32),
            pltpu.SemaphoreType.DMA,
            pltpu.SemaphoreType.DMA,
            pltpu.SemaphoreType.DMA,
            pltpu.SemaphoreType.DMA,
            pltpu.SemaphoreType.DMA,
            pltpu.SemaphoreType.DMA,
        ],
    )
    def sc_kernel(table_hbm, xs_hbm, ys_hbm, yneg_hbm, res_hbm,
                  xi_v, yi_v, ni_v, xy_v, neg0, neg1, neg2, neg3, res_v,
                  sem_xy, sem_g0, sem_g1, sem_g2, sem_g3, sem_w):
        wid = lax.axis_index("s") * NC + lax.axis_index("c")
        edge_base = wid * EDGES_PER_TILE
        negs = (neg0, neg1, neg2, neg3)
        sems = (sem_g0, sem_g1, sem_g2, sem_g3)

        pltpu.sync_copy(xs_hbm.at[pl.ds(edge_base, EDGES_PER_TILE)], xi_v)
        pltpu.sync_copy(ys_hbm.at[pl.ds(edge_base, EDGES_PER_TILE)], yi_v)
        pltpu.sync_copy(yneg_hbm.at[pl.ds(wid * NEG_PER_TILE, NEG_PER_TILE)],
                        ni_v)

        pltpu.async_copy(table_hbm.at[xi_v],
                         xy_v.at[pl.ds(0, EDGES_PER_TILE)], sem_xy)
        pltpu.async_copy(table_hbm.at[yi_v],
                         xy_v.at[pl.ds(EDGES_PER_TILE, EDGES_PER_TILE)],
                         sem_xy)

        def start_neg_gather(chunk, j):
            return pltpu.async_copy(
                table_hbm.at[ni_v.at[pl.ds(chunk * CHUNK_NEG_ROWS,
                                           CHUNK_NEG_ROWS)]],
                negs[j], sems[j])

        for j in range(NBUF):
            start_neg_gather(j, j)

        zeros = jnp.zeros((L,), jnp.float32)
        for r in range(RES_ROWS):
            for c in range(RES_LANES // L):
                res_v[r, pl.ds(c * L, L)] = zeros

        pltpu.make_async_copy(table_hbm.at[xi_v],
                              xy_v.at[pl.ds(0, EDGES_PER_TILE)],
                              sem_xy).wait()
        pltpu.make_async_copy(table_hbm.at[xi_v],
                              xy_v.at[pl.ds(0, EDGES_PER_TILE)],
                              sem_xy).wait()

        def compute_chunk(t, j, neg_v):
            for e in range(EDGES_PER_CHUNK):
                edge = (NBUF * t + j) * EDGES_PER_CHUNK + e
                res_row = j * EDGES_PER_CHUNK + e
                xq = [xy_v[edge, pl.ds(k * L, L)] for k in range(D // L)]
                for p in range(N_NEG):
                    prow = e * N_NEG + p
                    acc = None
                    for k in range(D // L):
                        dlt = xq[k] - neg_v[prow, pl.ds(k * L, L)]
                        sq = dlt * dlt
                        acc = sq if acc is None else acc + sq
                    res_v[res_row, pl.ds(p * L, L)] = acc
                acc = None
                for k in range(D // L):
                    dlt = xq[k] - xy_v[EDGES_PER_TILE + edge, pl.ds(k * L, L)]
                    sq = dlt * dlt
                    acc = sq if acc is None else acc + sq
                res_v[res_row, pl.ds(N_NEG * L, L)] = acc

        @pl.loop(0, CHUNKS_PER_TILE // NBUF)
        def _(t):
            @pl.when(t > 0)
            def _():
                pltpu.make_async_copy(
                    res_v, res_hbm.at[pl.ds(edge_base, RES_ROWS)], sem_w
                ).wait()

            for j in range(NBUF):
                chunk = NBUF * t + j
                pltpu.make_async_copy(
                    table_hbm.at[ni_v.at[pl.ds(chunk * CHUNK_NEG_ROWS,
                                               CHUNK_NEG_ROWS)]],
                    negs[j], sems[j]).wait()
                compute_chunk(t, j, negs[j])

                @pl.when(t < CHUNKS_PER_TILE // NBUF - 1)
                def _():
                    start_neg_gather(chunk + NBUF, j)

            pltpu.async_copy(
                res_v,
                res_hbm.at[pl.ds(edge_base + t * RES_ROWS, RES_ROWS)],
                sem_w)

        pltpu.make_async_copy(
            res_v, res_hbm.at[pl.ds(edge_base, RES_ROWS)], sem_w).wait()

    return sc_kernel(table, xs, ys, y_neg_flat)


def _tc_loss_body(r_ref, w_ref, o_ref):
    blk = r_ref[...]

    row_ids = lax.broadcasted_iota(jnp.int32, (RES_LANES, RES_LANES // L), 0)
    col_ids = lax.broadcasted_iota(jnp.int32, (RES_LANES, RES_LANES // L), 1)
    sel = (row_ids // L == col_ids).astype(jnp.float32)

    dmat = jax.lax.dot_general(
        blk, sel, (((1,), (0,)), ((), ())),
        preferred_element_type=jnp.float32)

    dneg = dmat[:, :N_NEG]
    dpos = dmat[:, N_NEG:N_NEG + 1]

    p_pos = jnp.clip(1.0 / (1.0 + 0.25 * dpos), 1e-12, 1.0 - 1e-12)
    p_neg = jnp.clip(1.0 / (1.0 + 0.25 * dneg), 1e-12, 0.99)
    t_pos = jnp.log(p_pos) * 20.0
    t_neg = jnp.log(1.0 - p_neg)

    loss_b = 7.0 * jnp.sum(t_neg, axis=1) + t_pos[:, 0]
    o_ref[...] = jnp.full((1, 1), -jnp.sum(w_ref[0, :] * loss_b), jnp.float32)


def _tc_loss(pair_partials, weights):
    out = pl.pallas_call(
        _tc_loss_body,
        out_shape=jax.ShapeDtypeStruct((1, 1), jnp.float32),
    )(pair_partials, weights)
    return out[0, 0]


@jax.jit
def kernel(logits, xs, ys, y_neg, sample_edge_weight):
    partials = _sc_pair_partials(logits, xs, ys, y_neg.reshape(-1))
    return _tc_loss(partials, sample_edge_weight.reshape(1, B))
